# baseline (speedup 1.0000x reference)
import functools

import jax
import jax.numpy as jnp
from jax import lax
from jax.experimental import pallas as pl
from jax.experimental.pallas import tpu as pltpu
from jax.experimental.pallas import tpu_sc as plsc

_INDICES = 1000000
_SIZE = 32
_BATCH = 16384
_SPLIT = 499968


def _build():
    info = plsc.get_sparse_core_info()
    num_cores, num_subcores = info.num_cores, info.num_subcores
    num_workers = num_cores * num_subcores
    b_per_w = _BATCH // num_workers
    lanes = info.num_lanes
    mesh = plsc.VectorSubcoreMesh(core_axis_name="c", subcore_axis_name="s")

    @functools.partial(
        pl.kernel,
        mesh=mesh,
        out_type=jax.ShapeDtypeStruct((_BATCH, _SIZE), jnp.float32),
        scratch_types=[
            pltpu.VMEM((b_per_w,), jnp.int32),
            pltpu.VMEM((b_per_w, _SIZE), jnp.float32),
            pltpu.SemaphoreType.DMA,
        ],
    )
    def gather_kernel(table_a, table_b, idx_hbm, out_hbm, idx_v, rows_v, sem):
        wid = lax.axis_index("s") * num_cores + lax.axis_index("c")
        base = wid * b_per_w
        pltpu.sync_copy(idx_hbm.at[pl.ds(base, b_per_w)], idx_v)

        def body(i, carry):
            vec = idx_v[pl.ds(i * lanes, lanes)]
            for j in range(lanes):
                r = vec[j]
                k = i * lanes + j

                @pl.when(r < _SPLIT)
                def _():
                    pltpu.async_copy(table_a.at[0, r], rows_v.at[k], sem)

                @pl.when(r >= _SPLIT)
                def _():
                    pltpu.async_copy(
                        table_b.at[0, r - _SPLIT], rows_v.at[k], sem
                    )

            return carry

        lax.fori_loop(0, b_per_w // lanes, body, 0)
        pltpu.make_async_copy(
            table_a.at[0, pl.ds(0, b_per_w)], rows_v, sem
        ).wait()
        pltpu.sync_copy(rows_v, out_hbm.at[pl.ds(base, b_per_w)])

    return gather_kernel


_gather = _build()


def kernel(x, update, weights):
    del update
    wa = weights[:_SPLIT].reshape(1, _SPLIT, _SIZE)
    wb = weights[_SPLIT:].reshape(1, _INDICES - _SPLIT, _SIZE)
    return _gather(wa, wb, x.astype(jnp.int32))


# speedup vs baseline: 1.2277x; 1.2277x over previous
import functools

import jax
import jax.numpy as jnp
from jax import lax
from jax.experimental import pallas as pl
from jax.experimental.pallas import tpu as pltpu
from jax.experimental.pallas import tpu_sc as plsc

_INDICES = 1000000
_SIZE = 32
_BATCH = 16384


def _build():
    info = plsc.get_sparse_core_info()
    num_cores, num_subcores = info.num_cores, info.num_subcores
    num_workers = num_cores * num_subcores
    b_per_w = _BATCH // num_workers
    lanes = info.num_lanes
    mesh = plsc.VectorSubcoreMesh(core_axis_name="c", subcore_axis_name="s")

    @functools.partial(
        pl.kernel,
        mesh=mesh,
        out_type=jax.ShapeDtypeStruct((_SIZE, _BATCH), jnp.float32),
        scratch_types=[
            pltpu.VMEM((b_per_w,), jnp.int32),
            pltpu.VMEM((b_per_w, _SIZE), jnp.float32),
            pltpu.VMEM((_SIZE, b_per_w), jnp.float32),
            pltpu.SemaphoreType.DMA,
        ],
        compiler_params=pltpu.CompilerParams(needs_layout_passes=False),
    )
    def gather_kernel(table_hbm, idx_hbm, out_hbm, idx_v, rows_v, cols_v, sem):
        wid = lax.axis_index("s") * num_cores + lax.axis_index("c")
        base = wid * b_per_w
        pltpu.sync_copy(idx_hbm.at[pl.ds(base, b_per_w)], idx_v)

        def body(i, carry):
            vec = idx_v[pl.ds(i * lanes, lanes)]
            for j in range(lanes):
                r = vec[j]
                k = i * lanes + j
                pltpu.async_copy(table_hbm.at[0, r], rows_v.at[k], sem)
            return carry

        lax.fori_loop(0, b_per_w // lanes, body, 0)
        pltpu.make_async_copy(
            table_hbm.at[0, pl.ds(0, b_per_w)], rows_v, sem
        ).wait()

        lane_iota = lax.iota(jnp.int32, lanes)

        def tbody(i, carry):
            kvec = lane_iota + i * lanes
            for c in range(_SIZE):
                cvec = jnp.full((lanes,), c, jnp.int32)
                vals = plsc.load_gather(rows_v, [kvec, cvec])
                cols_v[c, pl.ds(i * lanes, lanes)] = vals
            return carry

        lax.fori_loop(0, b_per_w // lanes, tbody, 0)
        pltpu.sync_copy(cols_v, out_hbm.at[:, pl.ds(base, b_per_w)])

    return gather_kernel


_gather = _build()


def kernel(x, update, weights):
    del update
    out_t = _gather(weights.reshape(1, _INDICES, _SIZE), x.astype(jnp.int32))
    return out_t.T


# speedup vs baseline: 1.2628x; 1.0286x over previous
import functools

import jax
import jax.numpy as jnp
from jax import lax
from jax.experimental import pallas as pl
from jax.experimental.pallas import tpu as pltpu
from jax.experimental.pallas import tpu_sc as plsc

_INDICES = 1000000
_SIZE = 32
_BATCH = 16384


def _build():
    info = plsc.get_sparse_core_info()
    num_cores, num_subcores = info.num_cores, info.num_subcores
    num_workers = num_cores * num_subcores
    b_per_w = _BATCH // num_workers
    lanes = info.num_lanes
    mesh = plsc.VectorSubcoreMesh(core_axis_name="c", subcore_axis_name="s")

    @functools.partial(
        pl.kernel,
        mesh=mesh,
        out_type=jax.ShapeDtypeStruct((_BATCH, _SIZE), jnp.float32),
        scratch_types=[
            pltpu.VMEM((b_per_w,), jnp.int32),
            pltpu.VMEM((b_per_w, _SIZE), jnp.float32),
            pltpu.SemaphoreType.DMA,
        ],
    )
    def gather_kernel(table_hbm, idx_hbm, out_hbm, idx_v, rows_v, sem):
        wid = lax.axis_index("s") * num_cores + lax.axis_index("c")
        base = wid * b_per_w
        pltpu.sync_copy(idx_hbm.at[pl.ds(base, b_per_w)], idx_v)

        def body(i, carry):
            vec = idx_v[pl.ds(i * lanes, lanes)]
            for j in range(lanes):
                r = vec[j]
                k = i * lanes + j
                pltpu.async_copy(table_hbm.at[0, r], rows_v.at[k], sem)
            return carry

        lax.fori_loop(0, b_per_w // lanes, body, 0)
        pltpu.make_async_copy(
            table_hbm.at[0, pl.ds(0, b_per_w)], rows_v, sem
        ).wait()
        pltpu.sync_copy(rows_v, out_hbm.at[pl.ds(base, b_per_w)])

    return gather_kernel


_gather = _build()


def kernel(x, update, weights):
    del update
    return _gather(weights.reshape(1, _INDICES, _SIZE), x.astype(jnp.int32))
